# Initial kernel scaffold; baseline (speedup 1.0000x reference)
import jax
import jax.numpy as jnp
from jax.experimental import pallas as pl


def kernel(x, edge_index, W1l, b1l, W1r, gamma1, beta1, W2l, b2l, W2r):
    raise NotImplementedError("write your pallas kernel here")


# speedup vs baseline: 6.2378x; 6.2378x over previous
import math

import jax
import jax.numpy as jnp
from jax import lax
from jax.experimental import pallas as pl
from jax.experimental.pallas import tpu as pltpu
from jax.experimental.pallas import tpu_sc as plsc

N_NODES = 10000
N_EDGES = 320000
D = 128
BN_EPS = 1e-5

NC = 2
NS = 16
NW = NC * NS
EPW = N_EDGES // NW
CH = 80
NCHUNK = EPW // CH
CNT_W = 16
STRIPE = 624
TAIL = N_NODES - NS * STRIPE

_BN_SCALE = 1.0 / math.sqrt(1.0 + BN_EPS)


def _mesh():
    return plsc.VectorSubcoreMesh(core_axis_name="c", subcore_axis_name="s",
                                  num_cores=NC, num_subcores=NS)


def _make_sc_seg():
    def body(y_hbm, src_hbm, dst_hbm, zrow_hbm, seg_out,
             acc, src_v, dst_v, rows_v, gsem):
        c = lax.axis_index("c")
        s = lax.axis_index("s")
        w = s * NC + c

        base = s * STRIPE
        pltpu.sync_copy(zrow_hbm.at[pl.ds(base, STRIPE)],
                        acc.at[pl.ds(base, STRIPE)])

        @pl.when(s == NS - 1)
        def _():
            pltpu.sync_copy(zrow_hbm.at[pl.ds(NS * STRIPE, TAIL)],
                            acc.at[pl.ds(NS * STRIPE, TAIL)])

        pltpu.sync_copy(src_hbm.at[w], src_v)
        pltpu.sync_copy(dst_hbm.at[w], dst_v)

        plsc.subcore_barrier()

        def chunk(k, carry):
            pltpu.async_copy(y_hbm.at[src_v.at[k]], rows_v, gsem).wait()
            pltpu.sync_copy(rows_v, acc.at[dst_v.at[k]], add=True)
            return carry

        lax.fori_loop(0, NCHUNK, chunk, 0)

        plsc.subcore_barrier()

        pltpu.sync_copy(acc.at[pl.ds(base, STRIPE)],
                        seg_out.at[c, pl.ds(base, STRIPE)])

        @pl.when(s == NS - 1)
        def _():
            pltpu.sync_copy(acc.at[pl.ds(NS * STRIPE, TAIL)],
                            seg_out.at[c, pl.ds(NS * STRIPE, TAIL)])

    return pl.kernel(
        body,
        out_type=jax.ShapeDtypeStruct((NC, N_NODES, D), jnp.float32),
        mesh=_mesh(),
        scratch_types=[
            pltpu.VMEM_SHARED((N_NODES, D), jnp.float32),
            pltpu.VMEM((NCHUNK, CH), jnp.int32),
            pltpu.VMEM((NCHUNK, CH), jnp.int32),
            pltpu.VMEM((CH, D), jnp.float32),
            pltpu.SemaphoreType.DMA,
        ],
    )


def _make_sc_cnt():
    def body(dst_hbm, zcnt_hbm, ones_hbm, cnt_out, cnt_acc, dst_v, ones_v):
        c = lax.axis_index("c")
        s = lax.axis_index("s")
        w = s * NC + c
        base = s * STRIPE

        pltpu.sync_copy(zcnt_hbm.at[pl.ds(base, STRIPE)],
                        cnt_acc.at[pl.ds(base, STRIPE)])

        @pl.when(s == NS - 1)
        def _():
            pltpu.sync_copy(zcnt_hbm.at[pl.ds(NS * STRIPE, TAIL)],
                            cnt_acc.at[pl.ds(NS * STRIPE, TAIL)])

        pltpu.sync_copy(dst_hbm.at[w], dst_v)
        pltpu.sync_copy(ones_hbm, ones_v)

        plsc.subcore_barrier()

        def chunk(k, carry):
            pltpu.sync_copy(ones_v, cnt_acc.at[dst_v.at[k]], add=True)
            return carry

        lax.fori_loop(0, NCHUNK, chunk, 0)

        plsc.subcore_barrier()

        pltpu.sync_copy(cnt_acc.at[pl.ds(base, STRIPE)],
                        cnt_out.at[c, pl.ds(base, STRIPE)])

        @pl.when(s == NS - 1)
        def _():
            pltpu.sync_copy(cnt_acc.at[pl.ds(NS * STRIPE, TAIL)],
                            cnt_out.at[c, pl.ds(NS * STRIPE, TAIL)])

    return pl.kernel(
        body,
        out_type=jax.ShapeDtypeStruct((NC, N_NODES, D), jnp.float32),
        mesh=_mesh(),
        scratch_types=[
            pltpu.VMEM_SHARED((N_NODES, D), jnp.float32),
            pltpu.VMEM((NCHUNK, CH), jnp.int32),
            pltpu.VMEM((CH, D), jnp.float32),
        ],
    )


_SC_CACHE = {}


def _sc_kernel(name):
    if name not in _SC_CACHE:
        _SC_CACHE[name] = _make_sc_seg() if name == "seg" else _make_sc_cnt()
    return _SC_CACHE[name]


_BM = 1000
_GRID = (N_NODES // _BM,)
_DOT = (((1,), (1,)), ((), ()))


def _row_spec(i):
    return (i, 0)


def _const_spec(i):
    return (0, 0)


def _plane_spec(i):
    return (0, i, 0)


def _dense1(x, W1l, W1r, b1l_row):
    def body(x_ref, wl_ref, wr_ref, b_ref, y_ref, z_ref):
        xb = x_ref[...]
        y_ref[...] = lax.dot_general(xb, wl_ref[...], _DOT,
                                     preferred_element_type=jnp.float32)
        z_ref[...] = lax.dot_general(xb, wr_ref[...], _DOT,
                                     preferred_element_type=jnp.float32) + b_ref[...]

    return pl.pallas_call(
        body,
        grid=_GRID,
        in_specs=[
            pl.BlockSpec((_BM, D), _row_spec),
            pl.BlockSpec((D, D), _const_spec),
            pl.BlockSpec((D, D), _const_spec),
            pl.BlockSpec((1, D), _const_spec),
        ],
        out_specs=[pl.BlockSpec((_BM, D), _row_spec)] * 2,
        out_shape=[jax.ShapeDtypeStruct((N_NODES, D), jnp.float32)] * 2,
    )(x, W1l, W1r, b1l_row)


def _dense2(seg1, cntp, z1, gamma_row, beta_row, W2l, W2r, b2l_row):
    def body(seg_ref, cnt_ref, z1_ref, g_ref, be_ref, wl_ref, wr_ref, b2_ref,
             y2_ref, z2_ref, cnt_out_ref):
        cnt = (cnt_ref[0] + cnt_ref[1])[:, 0:1]
        mean = (seg_ref[0] + seg_ref[1]) / jnp.clip(cnt, 1.0, None)
        h = (mean + z1_ref[...]) * (g_ref[...] * _BN_SCALE) + be_ref[...]
        h = jnp.maximum(h, 0.0)
        y2_ref[...] = lax.dot_general(h, wl_ref[...], _DOT,
                                      preferred_element_type=jnp.float32)
        z2_ref[...] = lax.dot_general(h, wr_ref[...], _DOT,
                                      preferred_element_type=jnp.float32) + b2_ref[...]
        cnt_out_ref[...] = jnp.broadcast_to(cnt, (_BM, CNT_W))

    return pl.pallas_call(
        body,
        grid=_GRID,
        in_specs=[
            pl.BlockSpec((NC, _BM, D), _plane_spec),
            pl.BlockSpec((NC, _BM, D), _plane_spec),
            pl.BlockSpec((_BM, D), _row_spec),
            pl.BlockSpec((1, D), _const_spec),
            pl.BlockSpec((1, D), _const_spec),
            pl.BlockSpec((D, D), _const_spec),
            pl.BlockSpec((D, D), _const_spec),
            pl.BlockSpec((1, D), _const_spec),
        ],
        out_specs=[
            pl.BlockSpec((_BM, D), _row_spec),
            pl.BlockSpec((_BM, D), _row_spec),
            pl.BlockSpec((_BM, CNT_W), _row_spec),
        ],
        out_shape=[
            jax.ShapeDtypeStruct((N_NODES, D), jnp.float32),
            jax.ShapeDtypeStruct((N_NODES, D), jnp.float32),
            jax.ShapeDtypeStruct((N_NODES, CNT_W), jnp.float32),
        ],
    )(seg1, cntp, z1, gamma_row, beta_row, W2l, W2r, b2l_row)


def _dense3(seg2, z2, cnt16_arr):
    def body(seg_ref, z2_ref, cnt_ref, o_ref):
        cnt = cnt_ref[:, 0:1]
        out = (seg_ref[0] + seg_ref[1]) / jnp.clip(cnt, 1.0, None) + z2_ref[...]
        m = jnp.max(out, axis=1, keepdims=True)
        e = jnp.exp(out - m)
        lse = jnp.log(jnp.sum(e, axis=1, keepdims=True)) + m
        o_ref[...] = out - lse

    return pl.pallas_call(
        body,
        grid=_GRID,
        in_specs=[
            pl.BlockSpec((NC, _BM, D), _plane_spec),
            pl.BlockSpec((_BM, D), _row_spec),
            pl.BlockSpec((_BM, CNT_W), _row_spec),
        ],
        out_specs=pl.BlockSpec((_BM, D), _row_spec),
        out_shape=jax.ShapeDtypeStruct((N_NODES, D), jnp.float32),
    )(seg2, z2, cnt16_arr)


def kernel(x, edge_index, W1l, b1l, W1r, gamma1, beta1, W2l, b2l, W2r):
    src = edge_index[0].reshape(NW, NCHUNK, CH)
    dst = edge_index[1].reshape(NW, NCHUNK, CH)
    zrow = jnp.zeros((N_NODES, D), jnp.float32)

    y1, z1 = _dense1(x, W1l, W1r, b1l.reshape(1, D))
    cntp = _sc_kernel("cnt")(dst, zrow, jnp.ones((CH, D), jnp.float32))
    seg1 = _sc_kernel("seg")(y1, src, dst, zrow)
    y2, z2, cnt16 = _dense2(seg1, cntp, z1, gamma1.reshape(1, D),
                            beta1.reshape(1, D), W2l, W2r, b2l.reshape(1, D))
    seg2 = _sc_kernel("seg")(y2, src, dst, zrow)
    return _dense3(seg2, z2, cnt16)


# speedup vs baseline: 9.1559x; 1.4678x over previous
import math

import jax
import jax.numpy as jnp
from jax import lax
from jax.experimental import pallas as pl
from jax.experimental.pallas import tpu as pltpu
from jax.experimental.pallas import tpu_sc as plsc

N_NODES = 10000
N_EDGES = 320000
D = 128
BN_EPS = 1e-5

NC = 2
NS = 16
NW = NC * NS
EPW = N_EDGES // NW
CH = 40
NCHUNK = EPW // CH
NBUF = 5
ROUNDS = NCHUNK // NBUF
CNT_W = 16
STRIPE = 624
TAIL = N_NODES - NS * STRIPE

_BN_SCALE = 1.0 / math.sqrt(1.0 + BN_EPS)


def _mesh():
    return plsc.VectorSubcoreMesh(core_axis_name="c", subcore_axis_name="s",
                                  num_cores=NC, num_subcores=NS)


def _make_sc_seg():
    def body(y_hbm, src_hbm, dst_hbm, zrow_hbm, seg_out, acc, *bufs):
        rows = bufs[:NBUF]
        si = (bufs[NBUF:2 * NBUF], bufs[2 * NBUF:3 * NBUF])
        di = (bufs[3 * NBUF:4 * NBUF], bufs[4 * NBUF:5 * NBUF])
        gsem = bufs[5 * NBUF:6 * NBUF]
        ssem = bufs[6 * NBUF:7 * NBUF]
        isem = (bufs[7 * NBUF:8 * NBUF], bufs[8 * NBUF:9 * NBUF])
        c = lax.axis_index("c")
        s = lax.axis_index("s")
        w = s * NC + c

        base = s * STRIPE
        pltpu.sync_copy(zrow_hbm.at[pl.ds(base, STRIPE)],
                        acc.at[pl.ds(base, STRIPE)])

        @pl.when(s == NS - 1)
        def _():
            pltpu.sync_copy(zrow_hbm.at[pl.ds(NS * STRIPE, TAIL)],
                            acc.at[pl.ds(NS * STRIPE, TAIL)])

        plsc.subcore_barrier()

        def istart(k, b, q):
            pltpu.async_copy(src_hbm.at[w, k], si[q][b], isem[q][b])
            pltpu.async_copy(dst_hbm.at[w, k], di[q][b], isem[q][b])

        def iwait(b, q):
            pltpu.make_async_copy(src_hbm.at[w, 0], si[q][b],
                                  isem[q][b]).wait()
            pltpu.make_async_copy(dst_hbm.at[w, 0], di[q][b],
                                  isem[q][b]).wait()

        def gstart(b, q):
            pltpu.async_copy(y_hbm.at[si[q][b]], rows[b], gsem[b])

        def gwait(b, q):
            pltpu.make_async_copy(y_hbm.at[si[q][b]], rows[b],
                                  gsem[b]).wait()

        def sstart(b, q):
            pltpu.async_copy(rows[b], acc.at[di[q][b]], ssem[b], add=True)

        def swait(b, q):
            pltpu.make_async_copy(rows[b], acc.at[di[q][b]],
                                  ssem[b]).wait()

        for b in range(NBUF):
            istart(b, b, 0)
        for b in range(NBUF):
            istart(NBUF + b, b, 1)
        for b in range(NBUF):
            iwait(b, 0)
            gstart(b, 0)

        def do_round(r, p):
            for b in range(NBUF):
                gwait(b, p)
                sstart(b, p)
            for b in range(NBUF):
                swait(b, p)

                @pl.when(r < ROUNDS - 2)
                def _(b=b, p=p, r=r):
                    istart(r * NBUF + 2 * NBUF + b, b, p)

                @pl.when(r < ROUNDS - 1)
                def _(b=b, p=p):
                    iwait(b, 1 - p)
                    gstart(b, 1 - p)

        def pair_body(t, carry):
            do_round(2 * t, 0)
            do_round(2 * t + 1, 1)
            return carry

        lax.fori_loop(0, ROUNDS // 2, pair_body, 0)

        plsc.subcore_barrier()

        pltpu.sync_copy(acc.at[pl.ds(base, STRIPE)],
                        seg_out.at[c, pl.ds(base, STRIPE)])

        @pl.when(s == NS - 1)
        def _():
            pltpu.sync_copy(acc.at[pl.ds(NS * STRIPE, TAIL)],
                            seg_out.at[c, pl.ds(NS * STRIPE, TAIL)])

    return pl.kernel(
        body,
        out_type=jax.ShapeDtypeStruct((NC, N_NODES, D), jnp.float32),
        mesh=_mesh(),
        scratch_types=(
            [pltpu.VMEM_SHARED((N_NODES, D), jnp.float32)]
            + [pltpu.VMEM((CH, D), jnp.float32)] * NBUF
            + [pltpu.VMEM((CH,), jnp.int32)] * (2 * NBUF)
            + [pltpu.VMEM((CH,), jnp.int32)] * (2 * NBUF)
            + [pltpu.SemaphoreType.DMA] * (4 * NBUF)
        ),
    )


def _make_sc_cnt():
    def body(dst_hbm, zcnt_hbm, ones_hbm, cnt_out, cnt_acc, dst_v, ones_v,
             *ssem):
        c = lax.axis_index("c")
        s = lax.axis_index("s")
        w = s * NC + c
        base = s * STRIPE

        pltpu.sync_copy(zcnt_hbm.at[pl.ds(base, STRIPE)],
                        cnt_acc.at[pl.ds(base, STRIPE)])

        @pl.when(s == NS - 1)
        def _():
            pltpu.sync_copy(zcnt_hbm.at[pl.ds(NS * STRIPE, TAIL)],
                            cnt_acc.at[pl.ds(NS * STRIPE, TAIL)])

        pltpu.sync_copy(dst_hbm.at[w], dst_v)
        pltpu.sync_copy(ones_hbm, ones_v)

        plsc.subcore_barrier()

        def round_body(r, carry):
            for b in range(NBUF):
                k = r * NBUF + b
                pltpu.async_copy(ones_v, cnt_acc.at[dst_v.at[k]], ssem[b],
                                 add=True)
            for b in range(NBUF):
                k = r * NBUF + b
                pltpu.make_async_copy(ones_v, cnt_acc.at[dst_v.at[k]],
                                      ssem[b]).wait()
            return carry

        lax.fori_loop(0, ROUNDS, round_body, 0)

        plsc.subcore_barrier()

        pltpu.sync_copy(cnt_acc.at[pl.ds(base, STRIPE)],
                        cnt_out.at[c, pl.ds(base, STRIPE)])

        @pl.when(s == NS - 1)
        def _():
            pltpu.sync_copy(cnt_acc.at[pl.ds(NS * STRIPE, TAIL)],
                            cnt_out.at[c, pl.ds(NS * STRIPE, TAIL)])

    return pl.kernel(
        body,
        out_type=jax.ShapeDtypeStruct((NC, N_NODES, D), jnp.float32),
        mesh=_mesh(),
        scratch_types=(
            [
                pltpu.VMEM_SHARED((N_NODES, D), jnp.float32),
                pltpu.VMEM((NCHUNK, CH), jnp.int32),
                pltpu.VMEM((CH, D), jnp.float32),
            ]
            + [pltpu.SemaphoreType.DMA] * NBUF
        ),
    )


_SC_CACHE = {}


def _sc_kernel(name):
    if name not in _SC_CACHE:
        _SC_CACHE[name] = _make_sc_seg() if name == "seg" else _make_sc_cnt()
    return _SC_CACHE[name]


_BM = 1000
_GRID = (N_NODES // _BM,)
_DOT = (((1,), (1,)), ((), ()))


def _row_spec(i):
    return (i, 0)


def _const_spec(i):
    return (0, 0)


def _plane_spec(i):
    return (0, i, 0)


def _dense1(x, W1l, W1r, b1l_row):
    def body(x_ref, wl_ref, wr_ref, b_ref, y_ref, z_ref):
        xb = x_ref[...]
        y_ref[...] = lax.dot_general(xb, wl_ref[...], _DOT,
                                     preferred_element_type=jnp.float32)
        z_ref[...] = lax.dot_general(xb, wr_ref[...], _DOT,
                                     preferred_element_type=jnp.float32) + b_ref[...]

    return pl.pallas_call(
        body,
        grid=_GRID,
        in_specs=[
            pl.BlockSpec((_BM, D), _row_spec),
            pl.BlockSpec((D, D), _const_spec),
            pl.BlockSpec((D, D), _const_spec),
            pl.BlockSpec((1, D), _const_spec),
        ],
        out_specs=[pl.BlockSpec((_BM, D), _row_spec)] * 2,
        out_shape=[jax.ShapeDtypeStruct((N_NODES, D), jnp.float32)] * 2,
    )(x, W1l, W1r, b1l_row)


def _dense2(seg1, cntp, z1, gamma_row, beta_row, W2l, W2r, b2l_row):
    def body(seg_ref, cnt_ref, z1_ref, g_ref, be_ref, wl_ref, wr_ref, b2_ref,
             y2_ref, z2_ref, cnt_out_ref):
        cnt = (cnt_ref[0] + cnt_ref[1])[:, 0:1]
        mean = (seg_ref[0] + seg_ref[1]) / jnp.clip(cnt, 1.0, None)
        h = (mean + z1_ref[...]) * (g_ref[...] * _BN_SCALE) + be_ref[...]
        h = jnp.maximum(h, 0.0)
        y2_ref[...] = lax.dot_general(h, wl_ref[...], _DOT,
                                      preferred_element_type=jnp.float32)
        z2_ref[...] = lax.dot_general(h, wr_ref[...], _DOT,
                                      preferred_element_type=jnp.float32) + b2_ref[...]
        cnt_out_ref[...] = jnp.broadcast_to(cnt, (_BM, CNT_W))

    return pl.pallas_call(
        body,
        grid=_GRID,
        in_specs=[
            pl.BlockSpec((NC, _BM, D), _plane_spec),
            pl.BlockSpec((NC, _BM, D), _plane_spec),
            pl.BlockSpec((_BM, D), _row_spec),
            pl.BlockSpec((1, D), _const_spec),
            pl.BlockSpec((1, D), _const_spec),
            pl.BlockSpec((D, D), _const_spec),
            pl.BlockSpec((D, D), _const_spec),
            pl.BlockSpec((1, D), _const_spec),
        ],
        out_specs=[
            pl.BlockSpec((_BM, D), _row_spec),
            pl.BlockSpec((_BM, D), _row_spec),
            pl.BlockSpec((_BM, CNT_W), _row_spec),
        ],
        out_shape=[
            jax.ShapeDtypeStruct((N_NODES, D), jnp.float32),
            jax.ShapeDtypeStruct((N_NODES, D), jnp.float32),
            jax.ShapeDtypeStruct((N_NODES, CNT_W), jnp.float32),
        ],
    )(seg1, cntp, z1, gamma_row, beta_row, W2l, W2r, b2l_row)


def _dense3(seg2, z2, cnt16_arr):
    def body(seg_ref, z2_ref, cnt_ref, o_ref):
        cnt = cnt_ref[:, 0:1]
        out = (seg_ref[0] + seg_ref[1]) / jnp.clip(cnt, 1.0, None) + z2_ref[...]
        m = jnp.max(out, axis=1, keepdims=True)
        e = jnp.exp(out - m)
        lse = jnp.log(jnp.sum(e, axis=1, keepdims=True)) + m
        o_ref[...] = out - lse

    return pl.pallas_call(
        body,
        grid=_GRID,
        in_specs=[
            pl.BlockSpec((NC, _BM, D), _plane_spec),
            pl.BlockSpec((_BM, D), _row_spec),
            pl.BlockSpec((_BM, CNT_W), _row_spec),
        ],
        out_specs=pl.BlockSpec((_BM, D), _row_spec),
        out_shape=jax.ShapeDtypeStruct((N_NODES, D), jnp.float32),
    )(seg2, z2, cnt16_arr)


def kernel(x, edge_index, W1l, b1l, W1r, gamma1, beta1, W2l, b2l, W2r):
    src = edge_index[0].reshape(NW, NCHUNK, CH)
    dst = edge_index[1].reshape(NW, NCHUNK, CH)
    zrow = jnp.zeros((N_NODES, D), jnp.float32)

    y1, z1 = _dense1(x, W1l, W1r, b1l.reshape(1, D))
    cntp = _sc_kernel("cnt")(dst, zrow, jnp.ones((CH, D), jnp.float32))
    seg1 = _sc_kernel("seg")(y1, src, dst, zrow)
    y2, z2, cnt16 = _dense2(seg1, cntp, z1, gamma1.reshape(1, D),
                            beta1.reshape(1, D), W2l, W2r, b2l.reshape(1, D))
    seg2 = _sc_kernel("seg")(y2, src, dst, zrow)
    return _dense3(seg2, z2, cnt16)


# speedup vs baseline: 9.1794x; 1.0026x over previous
import math

import jax
import jax.numpy as jnp
from jax import lax
from jax.experimental import pallas as pl
from jax.experimental.pallas import tpu as pltpu
from jax.experimental.pallas import tpu_sc as plsc

N_NODES = 10000
N_EDGES = 320000
D = 128
BN_EPS = 1e-5

NC = 2
NS = 16
NW = NC * NS
EPW = N_EDGES // NW
CH = 40
NCHUNK = EPW // CH
NBUF = 5
ROUNDS = NCHUNK // NBUF
CNT_W = 16
STRIPE = 624
TAIL = N_NODES - NS * STRIPE

_BN_SCALE = 1.0 / math.sqrt(1.0 + BN_EPS)


def _mesh():
    return plsc.VectorSubcoreMesh(core_axis_name="c", subcore_axis_name="s",
                                  num_cores=NC, num_subcores=NS)


def _make_sc_seg():
    def body(y_hbm, src_hbm, dst_hbm, zrow_hbm, seg_out, acc, *bufs):
        rows = bufs[:NBUF]
        si = (bufs[NBUF:2 * NBUF], bufs[2 * NBUF:3 * NBUF])
        di = (bufs[3 * NBUF:4 * NBUF], bufs[4 * NBUF:5 * NBUF])
        gsem = bufs[5 * NBUF:6 * NBUF]
        ssem = bufs[6 * NBUF:7 * NBUF]
        isem = (bufs[7 * NBUF:8 * NBUF], bufs[8 * NBUF:9 * NBUF])
        c = lax.axis_index("c")
        s = lax.axis_index("s")
        w = s * NC + c

        base = s * STRIPE
        pltpu.sync_copy(zrow_hbm.at[pl.ds(base, STRIPE)],
                        acc.at[pl.ds(base, STRIPE)])

        @pl.when(s == NS - 1)
        def _():
            pltpu.sync_copy(zrow_hbm.at[pl.ds(NS * STRIPE, TAIL)],
                            acc.at[pl.ds(NS * STRIPE, TAIL)])

        plsc.subcore_barrier()

        def istart(k, b, q):
            pltpu.async_copy(src_hbm.at[w, k], si[q][b], isem[q][b])
            pltpu.async_copy(dst_hbm.at[w, k], di[q][b], isem[q][b])

        def iwait(b, q):
            pltpu.make_async_copy(src_hbm.at[w, 0], si[q][b],
                                  isem[q][b]).wait()
            pltpu.make_async_copy(dst_hbm.at[w, 0], di[q][b],
                                  isem[q][b]).wait()

        def gstart(b, q):
            pltpu.async_copy(y_hbm.at[si[q][b]], rows[b], gsem[b])

        def gwait(b, q):
            pltpu.make_async_copy(y_hbm.at[si[q][b]], rows[b],
                                  gsem[b]).wait()

        def sstart(b, q):
            pltpu.async_copy(rows[b], acc.at[di[q][b]], ssem[b], add=True)

        def swait(b, q):
            pltpu.make_async_copy(rows[b], acc.at[di[q][b]],
                                  ssem[b]).wait()

        for b in range(NBUF):
            istart(b, b, 0)
        for b in range(NBUF):
            istart(NBUF + b, b, 1)
        for b in range(NBUF):
            iwait(b, 0)
            gstart(b, 0)

        def do_round(r, p):
            for b in range(NBUF):
                gwait(b, p)
                sstart(b, p)
            for b in range(NBUF):
                swait(b, p)

                @pl.when(r < ROUNDS - 2)
                def _(b=b, p=p, r=r):
                    istart(r * NBUF + 2 * NBUF + b, b, p)

                @pl.when(r < ROUNDS - 1)
                def _(b=b, p=p):
                    iwait(b, 1 - p)
                    gstart(b, 1 - p)

        def pair_body(t, carry):
            do_round(2 * t, 0)
            do_round(2 * t + 1, 1)
            return carry

        lax.fori_loop(0, ROUNDS // 2, pair_body, 0)

        plsc.subcore_barrier()

        pltpu.sync_copy(acc.at[pl.ds(base, STRIPE)],
                        seg_out.at[c, pl.ds(base, STRIPE)])

        @pl.when(s == NS - 1)
        def _():
            pltpu.sync_copy(acc.at[pl.ds(NS * STRIPE, TAIL)],
                            seg_out.at[c, pl.ds(NS * STRIPE, TAIL)])

    return pl.kernel(
        body,
        out_type=jax.ShapeDtypeStruct((NC, N_NODES, D), jnp.float32),
        mesh=_mesh(),
        scratch_types=(
            [pltpu.VMEM_SHARED((N_NODES, D), jnp.float32)]
            + [pltpu.VMEM((CH, D), jnp.float32)] * NBUF
            + [pltpu.VMEM((CH,), jnp.int32)] * (2 * NBUF)
            + [pltpu.VMEM((CH,), jnp.int32)] * (2 * NBUF)
            + [pltpu.SemaphoreType.DMA] * (4 * NBUF)
        ),
    )


def _make_sc_cnt():
    def body(dst_hbm, zrow_hbm, ones_hbm, cnt_out, acc, ones_v, *bufs):
        di = bufs[:NBUF]
        isem = bufs[NBUF:2 * NBUF]
        ssem = bufs[2 * NBUF:3 * NBUF]
        c = lax.axis_index("c")
        s = lax.axis_index("s")
        w = s * NC + c
        base = s * STRIPE

        pltpu.sync_copy(zrow_hbm.at[pl.ds(base, STRIPE)],
                        acc.at[pl.ds(base, STRIPE)])

        @pl.when(s == NS - 1)
        def _():
            pltpu.sync_copy(zrow_hbm.at[pl.ds(NS * STRIPE, TAIL)],
                            acc.at[pl.ds(NS * STRIPE, TAIL)])

        pltpu.sync_copy(ones_hbm, ones_v)
        plsc.subcore_barrier()

        def istart(k, b):
            pltpu.async_copy(dst_hbm.at[w, k], di[b], isem[b])

        def iwait(b):
            pltpu.make_async_copy(dst_hbm.at[w, 0], di[b], isem[b]).wait()

        def sstart(b):
            pltpu.async_copy(ones_v, acc.at[di[b]], ssem[b], add=True)

        def swait(b):
            pltpu.make_async_copy(ones_v, acc.at[di[b]], ssem[b]).wait()

        for b in range(NBUF):
            istart(b, b)

        def do_round(r, carry):
            for b in range(NBUF):
                iwait(b)
                sstart(b)
            for b in range(NBUF):
                swait(b)

                @pl.when(r < ROUNDS - 1)
                def _(b=b, r=r):
                    istart((r + 1) * NBUF + b, b)

            return carry

        lax.fori_loop(0, ROUNDS, do_round, 0)

        plsc.subcore_barrier()

        pltpu.sync_copy(acc.at[pl.ds(base, STRIPE)],
                        cnt_out.at[c, pl.ds(base, STRIPE)])

        @pl.when(s == NS - 1)
        def _():
            pltpu.sync_copy(acc.at[pl.ds(NS * STRIPE, TAIL)],
                            cnt_out.at[c, pl.ds(NS * STRIPE, TAIL)])

    return pl.kernel(
        body,
        out_type=jax.ShapeDtypeStruct((NC, N_NODES, D), jnp.float32),
        mesh=_mesh(),
        scratch_types=(
            [
                pltpu.VMEM_SHARED((N_NODES, D), jnp.float32),
                pltpu.VMEM((CH, D), jnp.float32),
            ]
            + [pltpu.VMEM((CH,), jnp.int32)] * NBUF
            + [pltpu.SemaphoreType.DMA] * (2 * NBUF)
        ),
    )


_SC_CACHE = {}


def _sc_kernel(name):
    if name not in _SC_CACHE:
        _SC_CACHE[name] = _make_sc_seg() if name == "seg" else _make_sc_cnt()
    return _SC_CACHE[name]


_BM = 1000
_GRID = (N_NODES // _BM,)
_DOT = (((1,), (1,)), ((), ()))


def _row_spec(i):
    return (i, 0)


def _const_spec(i):
    return (0, 0)


def _plane_spec(i):
    return (0, i, 0)


def _dense1(x, W1l, W1r, b1l_row):
    def body(x_ref, wl_ref, wr_ref, b_ref, y_ref, z_ref):
        xb = x_ref[...]
        y_ref[...] = lax.dot_general(xb, wl_ref[...], _DOT,
                                     preferred_element_type=jnp.float32)
        z_ref[...] = lax.dot_general(xb, wr_ref[...], _DOT,
                                     preferred_element_type=jnp.float32) + b_ref[...]

    return pl.pallas_call(
        body,
        grid=_GRID,
        in_specs=[
            pl.BlockSpec((_BM, D), _row_spec),
            pl.BlockSpec((D, D), _const_spec),
            pl.BlockSpec((D, D), _const_spec),
            pl.BlockSpec((1, D), _const_spec),
        ],
        out_specs=[pl.BlockSpec((_BM, D), _row_spec)] * 2,
        out_shape=[jax.ShapeDtypeStruct((N_NODES, D), jnp.float32)] * 2,
    )(x, W1l, W1r, b1l_row)


def _dense2(seg1, cntp, z1, gamma_row, beta_row, W2l, W2r, b2l_row):
    def body(seg_ref, cnt_ref, z1_ref, g_ref, be_ref, wl_ref, wr_ref, b2_ref,
             y2_ref, z2_ref):
        cnt = cnt_ref[0] + cnt_ref[1]
        mean = (seg_ref[0] + seg_ref[1]) / jnp.clip(cnt, 1.0, None)
        h = (mean + z1_ref[...]) * (g_ref[...] * _BN_SCALE) + be_ref[...]
        h = jnp.maximum(h, 0.0)
        y2_ref[...] = lax.dot_general(h, wl_ref[...], _DOT,
                                      preferred_element_type=jnp.float32)
        z2_ref[...] = lax.dot_general(h, wr_ref[...], _DOT,
                                      preferred_element_type=jnp.float32) + b2_ref[...]

    return pl.pallas_call(
        body,
        grid=_GRID,
        in_specs=[
            pl.BlockSpec((NC, _BM, D), _plane_spec),
            pl.BlockSpec((NC, _BM, 1), _plane_spec),
            pl.BlockSpec((_BM, D), _row_spec),
            pl.BlockSpec((1, D), _const_spec),
            pl.BlockSpec((1, D), _const_spec),
            pl.BlockSpec((D, D), _const_spec),
            pl.BlockSpec((D, D), _const_spec),
            pl.BlockSpec((1, D), _const_spec),
        ],
        out_specs=[
            pl.BlockSpec((_BM, D), _row_spec),
            pl.BlockSpec((_BM, D), _row_spec),
        ],
        out_shape=[
            jax.ShapeDtypeStruct((N_NODES, D), jnp.float32),
            jax.ShapeDtypeStruct((N_NODES, D), jnp.float32),
        ],
    )(seg1, cntp, z1, gamma_row, beta_row, W2l, W2r, b2l_row)


def _dense3(seg2, z2, cntp):
    def body(seg_ref, z2_ref, cnt_ref, o_ref):
        cnt = cnt_ref[0] + cnt_ref[1]
        out = (seg_ref[0] + seg_ref[1]) / jnp.clip(cnt, 1.0, None) + z2_ref[...]
        m = jnp.max(out, axis=1, keepdims=True)
        e = jnp.exp(out - m)
        lse = jnp.log(jnp.sum(e, axis=1, keepdims=True)) + m
        o_ref[...] = out - lse

    return pl.pallas_call(
        body,
        grid=_GRID,
        in_specs=[
            pl.BlockSpec((NC, _BM, D), _plane_spec),
            pl.BlockSpec((_BM, D), _row_spec),
            pl.BlockSpec((NC, _BM, 1), _plane_spec),
        ],
        out_specs=pl.BlockSpec((_BM, D), _row_spec),
        out_shape=jax.ShapeDtypeStruct((N_NODES, D), jnp.float32),
    )(seg2, z2, cntp)


def kernel(x, edge_index, W1l, b1l, W1r, gamma1, beta1, W2l, b2l, W2r):
    src = edge_index[0].reshape(NW, NCHUNK, CH)
    dst = edge_index[1].reshape(NW, NCHUNK, CH)
    zrow = jnp.zeros((N_NODES, D), jnp.float32)
    ones_ch = jnp.ones((CH, D), jnp.float32)

    y1, z1 = _dense1(x, W1l, W1r, b1l.reshape(1, D))
    cntp = _sc_kernel("cnt")(dst, zrow, ones_ch)
    cnt_bc = cntp[:, :, :1]
    seg1 = _sc_kernel("seg")(y1, src, dst, zrow)
    y2, z2 = _dense2(seg1, cnt_bc, z1, gamma1.reshape(1, D),
                     beta1.reshape(1, D), W2l, W2r, b2l.reshape(1, D))
    seg2 = _sc_kernel("seg")(y2, src, dst, zrow)
    return _dense3(seg2, z2, cnt_bc)
